# baseline (speedup 1.0000x reference)
import functools

import jax
import jax.numpy as jnp
from jax import lax
from jax.experimental import pallas as pl
from jax.experimental.pallas import tpu as pltpu
from jax.experimental.pallas import tpu_sc as plsc

D = 52
DP = 64
DE = 10
DEP = 16
K11 = 11
NSTEP = 3
N = 4000
NT = 8000
E1 = 32000
EP = 32768
ET = 2 * EP
NA = 8192
TRASH = 8100
CH = 128
NCHW = 16
NW = 32
RPS = NA // 16
RB = 1000
RBF = 200

f32 = jnp.float32
i32 = jnp.int32


def _pad2(w, r, c):
    return jnp.zeros((r, c), f32).at[: w.shape[0], : w.shape[1]].set(w)


def _embed_body(x_ref, w_ref, b_ref, o_ref):
    o_ref[...] = jnp.maximum(jnp.dot(x_ref[...], w_ref[0]) + b_ref[0], 0.0)


def _embed(x, w, b, rows, blk):
    nb = rows // blk
    c = x.shape[1]
    return pl.pallas_call(
        _embed_body,
        grid=(nb,),
        in_specs=[
            pl.BlockSpec((blk, c), lambda b_: (b_, 0)),
            pl.BlockSpec((1, c, w.shape[2]), lambda b_: (b_ // (nb // 2), 0, 0)),
            pl.BlockSpec((1, 1, w.shape[2]), lambda b_: (b_ // (nb // 2), 0, 0)),
        ],
        out_specs=pl.BlockSpec((blk, w.shape[2]), lambda b_: (b_, 0)),
        out_shape=jax.ShapeDtypeStruct((rows, w.shape[2]), f32),
    )(x, w, b)


def _msg_body(g_ref, ehx_ref, t_ref, s_ref, w_ref, o_ref):
    z = jnp.dot(g_ref[...], t_ref[...]) * jnp.dot(ehx_ref[...], s_ref[...])
    o_ref[...] = jnp.dot(z, w_ref[0])


def _msg(g, ehx, tmat, smat, wf):
    blk = 2048
    nb = ET // blk
    kd = K11 * DP
    return pl.pallas_call(
        _msg_body,
        grid=(nb,),
        in_specs=[
            pl.BlockSpec((blk, DP), lambda b_: (b_, 0)),
            pl.BlockSpec((blk, DEP), lambda b_: (b_, 0)),
            pl.BlockSpec((DP, kd), lambda b_: (0, 0)),
            pl.BlockSpec((DEP, kd), lambda b_: (0, 0)),
            pl.BlockSpec((1, kd, DP), lambda b_: (b_ // (nb // 2), 0, 0)),
        ],
        out_specs=pl.BlockSpec((blk, DP), lambda b_: (b_, 0)),
        out_shape=jax.ShapeDtypeStruct((ET, DP), f32),
    )(g, ehx, tmat, smat, wf)


def _upd_body(a0_ref, a1_ref, h_ref, wm_ref, wh_ref, b_ref, o_ref):
    m = jnp.maximum(a0_ref[0] + a1_ref[0], 0.0)
    o_ref[...] = (
        jnp.dot(m, wm_ref[0]) + jnp.dot(h_ref[...], wh_ref[0]) + b_ref[0]
    )


def _upd(agg2, h, wm, wh, b):
    nb = NT // RB
    return pl.pallas_call(
        _upd_body,
        grid=(nb,),
        in_specs=[
            pl.BlockSpec((1, RB, DP), lambda b_: (0, b_, 0)),
            pl.BlockSpec((1, RB, DP), lambda b_: (1, b_, 0)),
            pl.BlockSpec((RB, DP), lambda b_: (b_, 0)),
            pl.BlockSpec((1, DP, DP), lambda b_: (b_ // (nb // 2), 0, 0)),
            pl.BlockSpec((1, DP, DP), lambda b_: (b_ // (nb // 2), 0, 0)),
            pl.BlockSpec((1, 1, DP), lambda b_: (b_ // (nb // 2), 0, 0)),
        ],
        out_specs=pl.BlockSpec((RB, DP), lambda b_: (b_, 0)),
        out_shape=jax.ShapeDtypeStruct((NT, DP), f32),
    )(agg2, agg2, h, wm, wh, b)


def _final_body(hu_ref, xu_ref, hv_ref, xv_ref, bu_ref, bv_ref,
                im_ref, osu_ref, osv_ref, svn_ref, acc_ref):
    i = pl.program_id(0)

    @pl.when(i == 0)
    def _():
        sv = hv_ref[...] + xv_ref[...]
        nv = jnp.sqrt(jnp.sum(sv * sv, axis=1, keepdims=True))
        svn_ref[...] = sv / jnp.maximum(nv, 1e-12)

    su = hu_ref[...] + xu_ref[...]
    nu = jnp.sqrt(jnp.sum(su * su, axis=1, keepdims=True))
    su = su / jnp.maximum(nu, 1e-12)
    svn = svn_ref[...]
    raw = lax.dot_general(su, svn, (((1,), (1,)), ((), ())))
    im = jnp.where(bu_ref[...] == bv_ref[...], raw, 0.0)
    im_ref[...] = im
    osu_ref[:, 0, :] = su
    osu_ref[:, 1, :] = jnp.dot(im, svn)
    contrib = lax.dot_general(im, su, (((0,), (0,)), ((), ())))

    @pl.when(i == 0)
    def _():
        acc_ref[...] = contrib

    @pl.when(i > 0)
    def _():
        acc_ref[...] = acc_ref[...] + contrib

    @pl.when(i == pl.num_programs(0) - 1)
    def _():
        osv_ref[:, 0, :] = svn_ref[...]
        osv_ref[:, 1, :] = acc_ref[...]


def _final(h, xpad, bu, bv):
    nb = N // RBF
    return pl.pallas_call(
        _final_body,
        grid=(nb,),
        in_specs=[
            pl.BlockSpec((RBF, DP), lambda b_: (b_, 0)),
            pl.BlockSpec((RBF, DP), lambda b_: (b_, 0)),
            pl.BlockSpec((N, DP), lambda b_: (1, 0)),
            pl.BlockSpec((N, DP), lambda b_: (1, 0)),
            pl.BlockSpec((RBF, 1), lambda b_: (b_, 0)),
            pl.BlockSpec((1, N), lambda b_: (0, 0)),
        ],
        out_specs=[
            pl.BlockSpec((RBF, N), lambda b_: (b_, 0)),
            pl.BlockSpec((RBF, 2, DP), lambda b_: (b_, 0, 0)),
            pl.BlockSpec((N, 2, DP), lambda b_: (0, 0, 0)),
        ],
        out_shape=[
            jax.ShapeDtypeStruct((N, N), f32),
            jax.ShapeDtypeStruct((N, 2, DP), f32),
            jax.ShapeDtypeStruct((N, 2, DP), f32),
        ],
        scratch_shapes=[
            pltpu.VMEM((N, DP), f32),
            pltpu.VMEM((N, DP), f32),
        ],
    )(h, xpad[:N], h, xpad, bu, bv)


def _sc_mesh():
    return plsc.VectorSubcoreMesh(
        core_axis_name="c", subcore_axis_name="s", num_cores=2, num_subcores=16
    )


def _sc_gather(h_table, src2):
    @functools.partial(
        pl.kernel,
        out_type=jax.ShapeDtypeStruct((ET, DP), f32),
        mesh=_sc_mesh(),
        compiler_params=pltpu.CompilerParams(use_tc_tiling_on_sc=False),
        scratch_types=[
            pltpu.VMEM((NCHW, CH), i32),
            pltpu.VMEM((CH, DP), f32),
            pltpu.VMEM((CH, DP), f32),
            pltpu.SemaphoreType.DMA,
            pltpu.SemaphoreType.DMA,
        ],
    )
    def gk(h_hbm, src_hbm, out_hbm, idx_v, rows_a, rows_b, sem_a, sem_b):
        c = lax.axis_index("c")
        s = lax.axis_index("s")
        wid = s * 2 + c
        pltpu.sync_copy(src_hbm.at[pl.ds(wid * NCHW, NCHW)], idx_v)
        bufs = (rows_a, rows_b)
        sems = (sem_a, sem_b)
        cps = [None, None]
        cps[0] = pltpu.async_copy(h_hbm.at[idx_v.at[0]], bufs[0], sems[0])
        for j in range(NCHW):
            if j + 1 < NCHW:
                cps[(j + 1) % 2] = pltpu.async_copy(
                    h_hbm.at[idx_v.at[j + 1]], bufs[(j + 1) % 2], sems[(j + 1) % 2])
            cps[j % 2].wait()
            pltpu.sync_copy(bufs[j % 2], out_hbm.at[pl.ds((wid * NCHW + j) * CH, CH)])

    return gk(h_table, src2)


def _sc_scatter(msg, dst2, zeros_nt):
    @functools.partial(
        pl.kernel,
        out_type=jax.ShapeDtypeStruct((2, NA, DP), f32),
        mesh=_sc_mesh(),
        compiler_params=pltpu.CompilerParams(use_tc_tiling_on_sc=False),
        scratch_types=[
            pltpu.VMEM((NCHW, CH), i32),
            pltpu.VMEM((CH, DP), f32),
            pltpu.VMEM((CH, DP), f32),
            pltpu.VMEM_SHARED((NA, DP), f32),
            pltpu.SemaphoreType.DMA,
            pltpu.SemaphoreType.DMA,
        ],
    )
    def sk(msg_hbm, dst_hbm, z_hbm, out_hbm, idx_v, row_a, row_b, agg_sh,
           sem_a, sem_b):
        c = lax.axis_index("c")
        s = lax.axis_index("s")
        wid = s * 2 + c
        pltpu.sync_copy(z_hbm.at[pl.ds(s * RPS, RPS)], agg_sh.at[pl.ds(s * RPS, RPS)])
        pltpu.sync_copy(dst_hbm.at[pl.ds(wid * NCHW, NCHW)], idx_v)
        plsc.subcore_barrier()
        bufs = (row_a, row_b)
        sems = (sem_a, sem_b)
        cps = [None, None]
        cps[0] = pltpu.async_copy(
            msg_hbm.at[pl.ds(wid * NCHW * CH, CH)], bufs[0], sems[0])
        for j in range(NCHW):
            if j + 1 < NCHW:
                cps[(j + 1) % 2] = pltpu.async_copy(
                    msg_hbm.at[pl.ds((wid * NCHW + j + 1) * CH, CH)],
                    bufs[(j + 1) % 2], sems[(j + 1) % 2])
            cps[j % 2].wait()
            pltpu.sync_copy(bufs[j % 2], agg_sh.at[idx_v.at[j]], add=True)
        plsc.subcore_barrier()
        pltpu.sync_copy(
            agg_sh.at[pl.ds(s * RPS, RPS)],
            out_hbm.at[c].at[pl.ds(s * RPS, RPS)],
        )

    return sk(msg, dst2, zeros_nt)


def kernel(solute_x, solute_edge_index, solute_edge_attr, solvent_x,
           solvent_edge_index, solvent_edge_attr, solute_batch, solvent_batch,
           su_lin0_W, su_lin0_b, su_e1_W, su_e1_b, su_e2_W, su_e2_b,
           su_msg_W, su_msg_b, sv_lin0_W, sv_lin0_b, sv_e1_W, sv_e1_b,
           sv_e2_W, sv_e2_b, sv_msg_W, sv_msg_b):
    ei_u = solute_edge_index.astype(i32)
    ei_v = solvent_edge_index.astype(i32)
    src2 = (jnp.zeros((ET,), i32)
            .at[:E1].set(ei_u[0])
            .at[EP:EP + E1].set(ei_v[0] + N)).reshape(NW * NCHW, CH)
    dst2 = (jnp.full((ET,), TRASH, i32)
            .at[:E1].set(ei_u[1])
            .at[EP:EP + E1].set(ei_v[1] + N)).reshape(NW * NCHW, CH)

    xpad = jnp.zeros((NT, DP), f32).at[:, :D].set(
        jnp.concatenate([solute_x, solvent_x], axis=0))
    ea2 = (jnp.zeros((ET, DEP), f32)
           .at[:E1, :DE].set(solute_edge_attr)
           .at[EP:EP + E1, :DE].set(solvent_edge_attr))

    w0 = jnp.stack([_pad2(su_lin0_W, DP, DP), _pad2(sv_lin0_W, DP, DP)])
    b0 = jnp.stack([_pad2(su_lin0_b[None], 1, DP), _pad2(sv_lin0_b[None], 1, DP)])
    e1p = jnp.stack([_pad2(su_e1_W, DEP, DEP), _pad2(sv_e1_W, DEP, DEP)])
    b1p = jnp.stack([
        _pad2(su_e1_b[None], 1, DEP).at[0, DE].set(1.0),
        _pad2(sv_e1_b[None], 1, DEP).at[0, DE].set(1.0),
    ])

    def w3t(e2_W, e2_b):
        w3x = jnp.concatenate(
            [e2_W.reshape(DE, D, D), e2_b.reshape(1, D, D)], axis=0)
        pad = jnp.zeros((K11, DP, DP), f32).at[:, :D, :D].set(w3x)
        return pad.transpose(0, 2, 1).reshape(K11 * DP, DP)

    wf = jnp.stack([w3t(su_e2_W, su_e2_b), w3t(sv_e2_W, sv_e2_b)])
    tmat = jnp.tile(jnp.eye(DP, dtype=f32), (1, K11))
    smat = jnp.zeros((DEP, K11 * DP), f32)
    for _k in range(K11):
        smat = smat.at[_k, _k * DP:(_k + 1) * DP].set(1.0)
    wm = jnp.stack([_pad2(su_msg_W[:D], DP, DP), _pad2(sv_msg_W[:D], DP, DP)])
    wh = jnp.stack([_pad2(su_msg_W[D:], DP, DP), _pad2(sv_msg_W[D:], DP, DP)])
    bm = jnp.stack([_pad2(su_msg_b[None], 1, DP), _pad2(sv_msg_b[None], 1, DP)])
    zeros_na = jnp.zeros((NA, DP), f32)
    bu = solute_batch.astype(i32).reshape(N, 1)
    bv = solvent_batch.astype(i32).reshape(1, N)

    h = _embed(xpad, w0, b0, NT, RB)
    ehx = _embed(ea2, e1p, b1p, ET, 4096)
    for _ in range(NSTEP):
        g = _sc_gather(h, src2)
        msg = _msg(g, ehx, tmat, smat, wf)
        agg2 = _sc_scatter(msg, dst2, zeros_na)
        h = _upd(agg2, h, wm, wh, bm)

    im, osu, osv = _final(h, xpad, bu, bv)
    out_su = jnp.concatenate([osu[:, 0, :D], osu[:, 1, :D]], axis=1)
    out_sv = jnp.concatenate([osv[:, 0, :D], osv[:, 1, :D]], axis=1)
    return out_su, out_sv, im


# speedup vs baseline: 1.1323x; 1.1323x over previous
import functools

import jax
import jax.numpy as jnp
from jax import lax
from jax.experimental import pallas as pl
from jax.experimental.pallas import tpu as pltpu
from jax.experimental.pallas import tpu_sc as plsc

D = 52
DP = 64
DE = 10
DEP = 16
K11 = 11
KD = K11 * DP
NSTEP = 3
N = 4000
E1 = 32000
EP = 32768
NA = 4096
TRASH = 4050
CH = 128
NCHW = 8
NW = 32
RPS = NA // 16
RB = 1000
RBF = 200

f32 = jnp.float32
i32 = jnp.int32


def _pad2(w, r, c):
    return jnp.zeros((r, c), f32).at[: w.shape[0], : w.shape[1]].set(w)


def _embed_body(x_ref, w_ref, b_ref, o_ref):
    o_ref[...] = jnp.maximum(jnp.dot(x_ref[...], w_ref[...]) + b_ref[...], 0.0)


def _embed(x, w, b, blk):
    rows, c = x.shape
    nb = rows // blk
    return pl.pallas_call(
        _embed_body,
        grid=(nb,),
        in_specs=[
            pl.BlockSpec((blk, c), lambda b_: (b_, 0)),
            pl.BlockSpec((c, w.shape[1]), lambda b_: (0, 0)),
            pl.BlockSpec((1, w.shape[1]), lambda b_: (0, 0)),
        ],
        out_specs=pl.BlockSpec((blk, w.shape[1]), lambda b_: (b_, 0)),
        out_shape=jax.ShapeDtypeStruct((rows, w.shape[1]), f32),
    )(x, w, b)


def _msg_body(g_ref, ehx_ref, t_ref, s_ref, w_ref, o_ref):
    z = jnp.dot(g_ref[...], t_ref[...]) * jnp.dot(ehx_ref[...], s_ref[...])
    o_ref[...] = jnp.dot(z, w_ref[...])


def _msg(g, ehx, tmat, smat, wf):
    blk = 2048
    nb = EP // blk
    return pl.pallas_call(
        _msg_body,
        grid=(nb,),
        in_specs=[
            pl.BlockSpec((blk, DP), lambda b_: (b_, 0)),
            pl.BlockSpec((blk, DEP), lambda b_: (b_, 0)),
            pl.BlockSpec((DP, KD), lambda b_: (0, 0)),
            pl.BlockSpec((DEP, KD), lambda b_: (0, 0)),
            pl.BlockSpec((KD, DP), lambda b_: (0, 0)),
        ],
        out_specs=pl.BlockSpec((blk, DP), lambda b_: (b_, 0)),
        out_shape=jax.ShapeDtypeStruct((EP, DP), f32),
    )(g, ehx, tmat, smat, wf)


def _upd_body(a0_ref, a1_ref, h_ref, wm_ref, wh_ref, b_ref, o_ref):
    m = jnp.maximum(a0_ref[0] + a1_ref[0], 0.0)
    o_ref[...] = (
        jnp.dot(m, wm_ref[...]) + jnp.dot(h_ref[...], wh_ref[...]) + b_ref[...]
    )


def _upd(agg2, h, wm, wh, b):
    nb = N // RB
    return pl.pallas_call(
        _upd_body,
        grid=(nb,),
        in_specs=[
            pl.BlockSpec((1, RB, DP), lambda b_: (0, b_, 0)),
            pl.BlockSpec((1, RB, DP), lambda b_: (1, b_, 0)),
            pl.BlockSpec((RB, DP), lambda b_: (b_, 0)),
            pl.BlockSpec((DP, DP), lambda b_: (0, 0)),
            pl.BlockSpec((DP, DP), lambda b_: (0, 0)),
            pl.BlockSpec((1, DP), lambda b_: (0, 0)),
        ],
        out_specs=pl.BlockSpec((RB, DP), lambda b_: (b_, 0)),
        out_shape=jax.ShapeDtypeStruct((N, DP), f32),
    )(agg2, agg2, h, wm, wh, b)


def _final_body(hu_ref, xu_ref, hv_ref, xv_ref, bu_ref, bv_ref,
                im_ref, osu_ref, osv_ref, svn_ref, acc_ref):
    i = pl.program_id(0)

    @pl.when(i == 0)
    def _():
        sv = hv_ref[...] + xv_ref[...]
        nv = jnp.sqrt(jnp.sum(sv * sv, axis=1, keepdims=True))
        svn_ref[...] = sv / jnp.maximum(nv, 1e-12)

    su = hu_ref[...] + xu_ref[...]
    nu = jnp.sqrt(jnp.sum(su * su, axis=1, keepdims=True))
    su = su / jnp.maximum(nu, 1e-12)
    svn = svn_ref[...]
    raw = lax.dot_general(su, svn, (((1,), (1,)), ((), ())))
    im = jnp.where(bu_ref[...] == bv_ref[...], raw, 0.0)
    im_ref[...] = im
    osu_ref[:, 0, :] = su
    osu_ref[:, 1, :] = jnp.dot(im, svn)
    contrib = lax.dot_general(im, su, (((0,), (0,)), ((), ())))

    @pl.when(i == 0)
    def _():
        acc_ref[...] = contrib

    @pl.when(i > 0)
    def _():
        acc_ref[...] = acc_ref[...] + contrib

    @pl.when(i == pl.num_programs(0) - 1)
    def _():
        osv_ref[:, 0, :] = svn_ref[...]
        osv_ref[:, 1, :] = acc_ref[...]


def _final(hu, xu, hv, xv, bu, bv):
    nb = N // RBF
    return pl.pallas_call(
        _final_body,
        grid=(nb,),
        in_specs=[
            pl.BlockSpec((RBF, DP), lambda b_: (b_, 0)),
            pl.BlockSpec((RBF, DP), lambda b_: (b_, 0)),
            pl.BlockSpec((N, DP), lambda b_: (0, 0)),
            pl.BlockSpec((N, DP), lambda b_: (0, 0)),
            pl.BlockSpec((RBF, 1), lambda b_: (b_, 0)),
            pl.BlockSpec((1, N), lambda b_: (0, 0)),
        ],
        out_specs=[
            pl.BlockSpec((RBF, N), lambda b_: (b_, 0)),
            pl.BlockSpec((RBF, 2, DP), lambda b_: (b_, 0, 0)),
            pl.BlockSpec((N, 2, DP), lambda b_: (0, 0, 0)),
        ],
        out_shape=[
            jax.ShapeDtypeStruct((N, N), f32),
            jax.ShapeDtypeStruct((N, 2, DP), f32),
            jax.ShapeDtypeStruct((N, 2, DP), f32),
        ],
        scratch_shapes=[
            pltpu.VMEM((N, DP), f32),
            pltpu.VMEM((N, DP), f32),
        ],
    )(hu, xu, hv, xv, bu, bv)


def _sc_mesh():
    return plsc.VectorSubcoreMesh(
        core_axis_name="c", subcore_axis_name="s", num_cores=2, num_subcores=16
    )


def _sc_gather(h_table, src2):
    @functools.partial(
        pl.kernel,
        out_type=jax.ShapeDtypeStruct((EP, DP), f32),
        mesh=_sc_mesh(),
        compiler_params=pltpu.CompilerParams(use_tc_tiling_on_sc=False),
        scratch_types=[
            pltpu.VMEM((NCHW, CH), i32),
            pltpu.VMEM((CH, DP), f32),
            pltpu.VMEM((CH, DP), f32),
            pltpu.SemaphoreType.DMA,
            pltpu.SemaphoreType.DMA,
        ],
    )
    def gk(h_hbm, src_hbm, out_hbm, idx_v, rows_a, rows_b, sem_a, sem_b):
        c = lax.axis_index("c")
        s = lax.axis_index("s")
        wid = s * 2 + c
        pltpu.sync_copy(src_hbm.at[pl.ds(wid * NCHW, NCHW)], idx_v)
        bufs = (rows_a, rows_b)
        sems = (sem_a, sem_b)
        cps = [None, None]
        cps[0] = pltpu.async_copy(h_hbm.at[idx_v.at[0]], bufs[0], sems[0])
        for j in range(NCHW):
            if j + 1 < NCHW:
                cps[(j + 1) % 2] = pltpu.async_copy(
                    h_hbm.at[idx_v.at[j + 1]], bufs[(j + 1) % 2], sems[(j + 1) % 2])
            cps[j % 2].wait()
            pltpu.sync_copy(bufs[j % 2], out_hbm.at[pl.ds((wid * NCHW + j) * CH, CH)])

    return gk(h_table, src2)


def _sc_scatter(msg, dst2, zeros_na):
    @functools.partial(
        pl.kernel,
        out_type=jax.ShapeDtypeStruct((2, NA, DP), f32),
        mesh=_sc_mesh(),
        compiler_params=pltpu.CompilerParams(use_tc_tiling_on_sc=False),
        scratch_types=[
            pltpu.VMEM((NCHW, CH), i32),
            pltpu.VMEM((CH, DP), f32),
            pltpu.VMEM((CH, DP), f32),
            pltpu.VMEM_SHARED((NA, DP), f32),
            pltpu.SemaphoreType.DMA,
            pltpu.SemaphoreType.DMA,
        ],
    )
    def sk(msg_hbm, dst_hbm, z_hbm, out_hbm, idx_v, row_a, row_b, agg_sh,
           sem_a, sem_b):
        c = lax.axis_index("c")
        s = lax.axis_index("s")
        wid = s * 2 + c
        pltpu.sync_copy(z_hbm.at[pl.ds(s * RPS, RPS)], agg_sh.at[pl.ds(s * RPS, RPS)])
        pltpu.sync_copy(dst_hbm.at[pl.ds(wid * NCHW, NCHW)], idx_v)
        plsc.subcore_barrier()
        bufs = (row_a, row_b)
        sems = (sem_a, sem_b)
        cps = [None, None]
        cps[0] = pltpu.async_copy(
            msg_hbm.at[pl.ds(wid * NCHW * CH, CH)], bufs[0], sems[0])
        for j in range(NCHW):
            if j + 1 < NCHW:
                cps[(j + 1) % 2] = pltpu.async_copy(
                    msg_hbm.at[pl.ds((wid * NCHW + j + 1) * CH, CH)],
                    bufs[(j + 1) % 2], sems[(j + 1) % 2])
            cps[j % 2].wait()
            pltpu.sync_copy(bufs[j % 2], agg_sh.at[idx_v.at[j]], add=True)
        plsc.subcore_barrier()
        pltpu.sync_copy(
            agg_sh.at[pl.ds(s * RPS, RPS)],
            out_hbm.at[c].at[pl.ds(s * RPS, RPS)],
        )

    return sk(msg, dst2, zeros_na)


def kernel(solute_x, solute_edge_index, solute_edge_attr, solvent_x,
           solvent_edge_index, solvent_edge_attr, solute_batch, solvent_batch,
           su_lin0_W, su_lin0_b, su_e1_W, su_e1_b, su_e2_W, su_e2_b,
           su_msg_W, su_msg_b, sv_lin0_W, sv_lin0_b, sv_e1_W, sv_e1_b,
           sv_e2_W, sv_e2_b, sv_msg_W, sv_msg_b):
    def idxpair(ei):
        ei = ei.astype(i32)
        src = jnp.zeros((EP,), i32).at[:E1].set(ei[0]).reshape(NW * NCHW, CH)
        dst = jnp.full((EP,), TRASH, i32).at[:E1].set(ei[1]).reshape(NW * NCHW, CH)
        return src, dst

    src_u, dst_u = idxpair(solute_edge_index)
    src_v, dst_v = idxpair(solvent_edge_index)

    def padx(x):
        return jnp.zeros((N, DP), f32).at[:, :D].set(x)

    def padea(ea):
        return jnp.zeros((EP, DEP), f32).at[:E1, :DE].set(ea)

    xu, xv = padx(solute_x), padx(solvent_x)
    ea_u, ea_v = padea(solute_edge_attr), padea(solvent_edge_attr)

    def wset(lin0_W, lin0_b, e1_W, e1_b, e2_W, e2_b, msg_W, msg_b):
        w3x = jnp.concatenate(
            [e2_W.reshape(DE, D, D), e2_b.reshape(1, D, D)], axis=0)
        wf = (jnp.zeros((K11, DP, DP), f32).at[:, :D, :D].set(w3x)
              .transpose(0, 2, 1).reshape(KD, DP))
        return dict(
            w0=_pad2(lin0_W, DP, DP), b0=_pad2(lin0_b[None], 1, DP),
            e1=_pad2(e1_W, DEP, DEP),
            b1=_pad2(e1_b[None], 1, DEP).at[0, DE].set(1.0),
            wf=wf,
            wm=_pad2(msg_W[:D], DP, DP), wh=_pad2(msg_W[D:], DP, DP),
            bm=_pad2(msg_b[None], 1, DP),
        )

    wu = wset(su_lin0_W, su_lin0_b, su_e1_W, su_e1_b, su_e2_W, su_e2_b,
              su_msg_W, su_msg_b)
    wv = wset(sv_lin0_W, sv_lin0_b, sv_e1_W, sv_e1_b, sv_e2_W, sv_e2_b,
              sv_msg_W, sv_msg_b)

    tmat = jnp.tile(jnp.eye(DP, dtype=f32), (1, K11))
    smat = jnp.zeros((DEP, KD), f32)
    for _k in range(K11):
        smat = smat.at[_k, _k * DP:(_k + 1) * DP].set(1.0)
    zeros_na = jnp.zeros((NA, DP), f32)
    bu = solute_batch.astype(i32).reshape(N, 1)
    bv = solvent_batch.astype(i32).reshape(1, N)

    hu = _embed(xu, wu["w0"], wu["b0"], RB)
    hv = _embed(xv, wv["w0"], wv["b0"], RB)
    ehx_u = _embed(ea_u, wu["e1"], wu["b1"], 4096)
    ehx_v = _embed(ea_v, wv["e1"], wv["b1"], 4096)
    for _ in range(NSTEP):
        g_u = _sc_gather(hu, src_u)
        g_v = _sc_gather(hv, src_v)
        m_u = _msg(g_u, ehx_u, tmat, smat, wu["wf"])
        m_v = _msg(g_v, ehx_v, tmat, smat, wv["wf"])
        a_u = _sc_scatter(m_u, dst_u, zeros_na)
        a_v = _sc_scatter(m_v, dst_v, zeros_na)
        hu = _upd(a_u, hu, wu["wm"], wu["wh"], wu["bm"])
        hv = _upd(a_v, hv, wv["wm"], wv["wh"], wv["bm"])

    im, osu, osv = _final(hu, xu, hv, xv, bu, bv)
    out_su = jnp.concatenate([osu[:, 0, :D], osu[:, 1, :D]], axis=1)
    out_sv = jnp.concatenate([osv[:, 0, :D], osv[:, 1, :D]], axis=1)
    return out_su, out_sv, im


# speedup vs baseline: 1.2481x; 1.1022x over previous
import functools

import jax
import jax.numpy as jnp
from jax import lax
from jax.experimental import pallas as pl
from jax.experimental.pallas import tpu as pltpu
from jax.experimental.pallas import tpu_sc as plsc

D = 52
DP = 64
DE = 10
DEP = 16
K11 = 11
KD = K11 * DP
NSTEP = 3
N = 4000
E1 = 32000
EP = 32768
NA = 4096
TRASH = 4050
CH = 128
NCHW = 8
NW = 32
RPS = NA // 16
RB = 1000
RBF = 200

f32 = jnp.float32
i32 = jnp.int32


def _pad2(w, r, c):
    return jnp.zeros((r, c), f32).at[: w.shape[0], : w.shape[1]].set(w)


def _embed_body(x_ref, w_ref, b_ref, o_ref):
    o_ref[...] = jnp.maximum(jnp.dot(x_ref[...], w_ref[...]) + b_ref[...], 0.0)


def _embed(x, w, b, blk):
    rows, c = x.shape
    nb = rows // blk
    return pl.pallas_call(
        _embed_body,
        grid=(nb,),
        in_specs=[
            pl.BlockSpec((blk, c), lambda b_: (b_, 0)),
            pl.BlockSpec((c, w.shape[1]), lambda b_: (0, 0)),
            pl.BlockSpec((1, w.shape[1]), lambda b_: (0, 0)),
        ],
        out_specs=pl.BlockSpec((blk, w.shape[1]), lambda b_: (b_, 0)),
        out_shape=jax.ShapeDtypeStruct((rows, w.shape[1]), f32),
    )(x, w, b)


def _msg_body(g_ref, ehx_ref, t_ref, s_ref, w_ref, o_ref):
    z = jnp.dot(g_ref[...], t_ref[...]) * jnp.dot(ehx_ref[...], s_ref[...])
    o_ref[...] = jnp.dot(z, w_ref[...])


def _msg(g, ehx, tmat, smat, wf):
    blk = 2048
    nb = EP // blk
    return pl.pallas_call(
        _msg_body,
        grid=(nb,),
        in_specs=[
            pl.BlockSpec((blk, DP), lambda b_: (b_, 0)),
            pl.BlockSpec((blk, DEP), lambda b_: (b_, 0)),
            pl.BlockSpec((DP, KD), lambda b_: (0, 0)),
            pl.BlockSpec((DEP, KD), lambda b_: (0, 0)),
            pl.BlockSpec((KD, DP), lambda b_: (0, 0)),
        ],
        out_specs=pl.BlockSpec((blk, DP), lambda b_: (b_, 0)),
        out_shape=jax.ShapeDtypeStruct((EP, DP), f32),
    )(g, ehx, tmat, smat, wf)


def _upd_body(a0_ref, a1_ref, h_ref, wm_ref, wh_ref, b_ref, o_ref):
    m = jnp.maximum(a0_ref[0] + a1_ref[0], 0.0)
    o_ref[...] = (
        jnp.dot(m, wm_ref[...]) + jnp.dot(h_ref[...], wh_ref[...]) + b_ref[...]
    )


def _upd(agg2, h, wm, wh, b):
    nb = N // RB
    return pl.pallas_call(
        _upd_body,
        grid=(nb,),
        in_specs=[
            pl.BlockSpec((1, RB, DP), lambda b_: (0, b_, 0)),
            pl.BlockSpec((1, RB, DP), lambda b_: (1, b_, 0)),
            pl.BlockSpec((RB, DP), lambda b_: (b_, 0)),
            pl.BlockSpec((DP, DP), lambda b_: (0, 0)),
            pl.BlockSpec((DP, DP), lambda b_: (0, 0)),
            pl.BlockSpec((1, DP), lambda b_: (0, 0)),
        ],
        out_specs=pl.BlockSpec((RB, DP), lambda b_: (b_, 0)),
        out_shape=jax.ShapeDtypeStruct((N, DP), f32),
    )(agg2, agg2, h, wm, wh, b)


def _final_body(hu_ref, xu_ref, hv_ref, xv_ref, bu_ref, bv_ref,
                im_ref, osu_ref, osv_ref, svn_ref, acc_ref):
    i = pl.program_id(0)

    @pl.when(i == 0)
    def _():
        sv = hv_ref[...] + xv_ref[...]
        nv = jnp.sqrt(jnp.sum(sv * sv, axis=1, keepdims=True))
        svn_ref[...] = sv / jnp.maximum(nv, 1e-12)

    su = hu_ref[...] + xu_ref[...]
    nu = jnp.sqrt(jnp.sum(su * su, axis=1, keepdims=True))
    su = su / jnp.maximum(nu, 1e-12)
    svn = svn_ref[...]
    raw = lax.dot_general(su, svn, (((1,), (1,)), ((), ())))
    im = jnp.where(bu_ref[...] == bv_ref[...], raw, 0.0)
    im_ref[...] = im
    osu_ref[:, 0, :] = su
    osu_ref[:, 1, :] = jnp.dot(im, svn)
    contrib = lax.dot_general(im, su, (((0,), (0,)), ((), ())))

    @pl.when(i == 0)
    def _():
        acc_ref[...] = contrib

    @pl.when(i > 0)
    def _():
        acc_ref[...] = acc_ref[...] + contrib

    @pl.when(i == pl.num_programs(0) - 1)
    def _():
        osv_ref[:, 0, :] = svn_ref[...]
        osv_ref[:, 1, :] = acc_ref[...]


def _final(hu, xu, hv, xv, bu, bv):
    nb = N // RBF
    return pl.pallas_call(
        _final_body,
        grid=(nb,),
        in_specs=[
            pl.BlockSpec((RBF, DP), lambda b_: (b_, 0)),
            pl.BlockSpec((RBF, DP), lambda b_: (b_, 0)),
            pl.BlockSpec((N, DP), lambda b_: (0, 0)),
            pl.BlockSpec((N, DP), lambda b_: (0, 0)),
            pl.BlockSpec((RBF, 1), lambda b_: (b_, 0)),
            pl.BlockSpec((1, N), lambda b_: (0, 0)),
        ],
        out_specs=[
            pl.BlockSpec((RBF, N), lambda b_: (b_, 0)),
            pl.BlockSpec((RBF, 2, DP), lambda b_: (b_, 0, 0)),
            pl.BlockSpec((N, 2, DP), lambda b_: (0, 0, 0)),
        ],
        out_shape=[
            jax.ShapeDtypeStruct((N, N), f32),
            jax.ShapeDtypeStruct((N, 2, DP), f32),
            jax.ShapeDtypeStruct((N, 2, DP), f32),
        ],
        scratch_shapes=[
            pltpu.VMEM((N, DP), f32),
            pltpu.VMEM((N, DP), f32),
        ],
    )(hu, xu, hv, xv, bu, bv)


def _sc_mesh():
    return plsc.VectorSubcoreMesh(
        core_axis_name="c", subcore_axis_name="s", num_cores=2, num_subcores=16
    )


def _sc_gather(h_table, src2):
    @functools.partial(
        pl.kernel,
        out_type=jax.ShapeDtypeStruct((EP, DP), f32),
        mesh=_sc_mesh(),
        compiler_params=pltpu.CompilerParams(use_tc_tiling_on_sc=False),
        scratch_types=[
            pltpu.VMEM((NCHW, CH), i32),
            pltpu.VMEM((CH, DP), f32),
            pltpu.VMEM((CH, DP), f32),
            pltpu.VMEM_SHARED((N, DP), f32),
            pltpu.SemaphoreType.DMA,
            pltpu.SemaphoreType.DMA,
        ],
    )
    def gk(h_hbm, src_hbm, out_hbm, idx_v, rows_a, rows_b, tab_sh,
           sem_a, sem_b):
        c = lax.axis_index("c")
        s = lax.axis_index("s")
        wid = s * 2 + c

        @pl.when(s < 15)
        def _():
            pltpu.sync_copy(h_hbm.at[pl.ds(s * 256, 256)],
                            tab_sh.at[pl.ds(s * 256, 256)])

        @pl.when(s == 15)
        def _():
            pltpu.sync_copy(h_hbm.at[pl.ds(3840, 160)],
                            tab_sh.at[pl.ds(3840, 160)])

        pltpu.sync_copy(src_hbm.at[pl.ds(wid * NCHW, NCHW)], idx_v)
        plsc.subcore_barrier()
        bufs = (rows_a, rows_b)
        sems = (sem_a, sem_b)
        cps = [None, None]
        cps[0] = pltpu.async_copy(tab_sh.at[idx_v.at[0]], bufs[0], sems[0])
        for j in range(NCHW):
            if j + 1 < NCHW:
                cps[(j + 1) % 2] = pltpu.async_copy(
                    tab_sh.at[idx_v.at[j + 1]], bufs[(j + 1) % 2],
                    sems[(j + 1) % 2])
            cps[j % 2].wait()
            pltpu.sync_copy(bufs[j % 2],
                            out_hbm.at[pl.ds((wid * NCHW + j) * CH, CH)])

    return gk(h_table, src2)


def _sc_scatter(msg, dst2, zeros_na):
    @functools.partial(
        pl.kernel,
        out_type=jax.ShapeDtypeStruct((2, NA, DP), f32),
        mesh=_sc_mesh(),
        compiler_params=pltpu.CompilerParams(use_tc_tiling_on_sc=False),
        scratch_types=[
            pltpu.VMEM((NCHW, CH), i32),
            pltpu.VMEM((CH, DP), f32),
            pltpu.VMEM((CH, DP), f32),
            pltpu.VMEM_SHARED((NA, DP), f32),
            pltpu.SemaphoreType.DMA,
            pltpu.SemaphoreType.DMA,
        ],
    )
    def sk(msg_hbm, dst_hbm, z_hbm, out_hbm, idx_v, row_a, row_b, agg_sh,
           sem_a, sem_b):
        c = lax.axis_index("c")
        s = lax.axis_index("s")
        wid = s * 2 + c
        pltpu.sync_copy(z_hbm.at[pl.ds(s * RPS, RPS)], agg_sh.at[pl.ds(s * RPS, RPS)])
        pltpu.sync_copy(dst_hbm.at[pl.ds(wid * NCHW, NCHW)], idx_v)
        plsc.subcore_barrier()
        bufs = (row_a, row_b)
        sems = (sem_a, sem_b)
        cps = [None, None]
        cps[0] = pltpu.async_copy(
            msg_hbm.at[pl.ds(wid * NCHW * CH, CH)], bufs[0], sems[0])
        for j in range(NCHW):
            if j + 1 < NCHW:
                cps[(j + 1) % 2] = pltpu.async_copy(
                    msg_hbm.at[pl.ds((wid * NCHW + j + 1) * CH, CH)],
                    bufs[(j + 1) % 2], sems[(j + 1) % 2])
            cps[j % 2].wait()
            pltpu.sync_copy(bufs[j % 2], agg_sh.at[idx_v.at[j]], add=True)
        plsc.subcore_barrier()
        pltpu.sync_copy(
            agg_sh.at[pl.ds(s * RPS, RPS)],
            out_hbm.at[c].at[pl.ds(s * RPS, RPS)],
        )

    return sk(msg, dst2, zeros_na)


def kernel(solute_x, solute_edge_index, solute_edge_attr, solvent_x,
           solvent_edge_index, solvent_edge_attr, solute_batch, solvent_batch,
           su_lin0_W, su_lin0_b, su_e1_W, su_e1_b, su_e2_W, su_e2_b,
           su_msg_W, su_msg_b, sv_lin0_W, sv_lin0_b, sv_e1_W, sv_e1_b,
           sv_e2_W, sv_e2_b, sv_msg_W, sv_msg_b):
    def idxpair(ei):
        ei = ei.astype(i32)
        src = jnp.zeros((EP,), i32).at[:E1].set(ei[0]).reshape(NW * NCHW, CH)
        dst = jnp.full((EP,), TRASH, i32).at[:E1].set(ei[1]).reshape(NW * NCHW, CH)
        return src, dst

    src_u, dst_u = idxpair(solute_edge_index)
    src_v, dst_v = idxpair(solvent_edge_index)

    def padx(x):
        return jnp.zeros((N, DP), f32).at[:, :D].set(x)

    def padea(ea):
        return jnp.zeros((EP, DEP), f32).at[:E1, :DE].set(ea)

    xu, xv = padx(solute_x), padx(solvent_x)
    ea_u, ea_v = padea(solute_edge_attr), padea(solvent_edge_attr)

    def wset(lin0_W, lin0_b, e1_W, e1_b, e2_W, e2_b, msg_W, msg_b):
        w3x = jnp.concatenate(
            [e2_W.reshape(DE, D, D), e2_b.reshape(1, D, D)], axis=0)
        wf = (jnp.zeros((K11, DP, DP), f32).at[:, :D, :D].set(w3x)
              .transpose(0, 2, 1).reshape(KD, DP))
        return dict(
            w0=_pad2(lin0_W, DP, DP), b0=_pad2(lin0_b[None], 1, DP),
            e1=_pad2(e1_W, DEP, DEP),
            b1=_pad2(e1_b[None], 1, DEP).at[0, DE].set(1.0),
            wf=wf,
            wm=_pad2(msg_W[:D], DP, DP), wh=_pad2(msg_W[D:], DP, DP),
            bm=_pad2(msg_b[None], 1, DP),
        )

    wu = wset(su_lin0_W, su_lin0_b, su_e1_W, su_e1_b, su_e2_W, su_e2_b,
              su_msg_W, su_msg_b)
    wv = wset(sv_lin0_W, sv_lin0_b, sv_e1_W, sv_e1_b, sv_e2_W, sv_e2_b,
              sv_msg_W, sv_msg_b)

    tmat = jnp.tile(jnp.eye(DP, dtype=f32), (1, K11))
    smat = jnp.zeros((DEP, KD), f32)
    for _k in range(K11):
        smat = smat.at[_k, _k * DP:(_k + 1) * DP].set(1.0)
    zeros_na = jnp.zeros((NA, DP), f32)
    bu = solute_batch.astype(i32).reshape(N, 1)
    bv = solvent_batch.astype(i32).reshape(1, N)

    hu = _embed(xu, wu["w0"], wu["b0"], RB)
    hv = _embed(xv, wv["w0"], wv["b0"], RB)
    ehx_u = _embed(ea_u, wu["e1"], wu["b1"], 4096)
    ehx_v = _embed(ea_v, wv["e1"], wv["b1"], 4096)
    for _ in range(NSTEP):
        g_u = _sc_gather(hu, src_u)
        g_v = _sc_gather(hv, src_v)
        m_u = _msg(g_u, ehx_u, tmat, smat, wu["wf"])
        m_v = _msg(g_v, ehx_v, tmat, smat, wv["wf"])
        a_u = _sc_scatter(m_u, dst_u, zeros_na)
        a_v = _sc_scatter(m_v, dst_v, zeros_na)
        hu = _upd(a_u, hu, wu["wm"], wu["wh"], wu["bm"])
        hv = _upd(a_v, hv, wv["wm"], wv["wh"], wv["bm"])

    im, osu, osv = _final(hu, xu, hv, xv, bu, bv)
    out_su = jnp.concatenate([osu[:, 0, :D], osu[:, 1, :D]], axis=1)
    out_sv = jnp.concatenate([osv[:, 0, :D], osv[:, 1, :D]], axis=1)
    return out_su, out_sv, im


# speedup vs baseline: 1.4809x; 1.1866x over previous
import functools

import jax
import jax.numpy as jnp
from jax import lax
from jax.experimental import pallas as pl
from jax.experimental.pallas import tpu as pltpu
from jax.experimental.pallas import tpu_sc as plsc

D = 52
DP = 64
DE = 10
DEP = 16
K11 = 11
KD = K11 * DP
NSTEP = 3
N = 4000
E1 = 32000
EP = 32768
NA = 4096
TRASH = 4050
CH = 128
NCHW = 8
NW = 32
RPS = NA // 16
RB = 1000
RBF = 200

f32 = jnp.float32
i32 = jnp.int32


def _pad2(w, r, c):
    return jnp.zeros((r, c), f32).at[: w.shape[0], : w.shape[1]].set(w)


def _embed_body(x_ref, w_ref, b_ref, o_ref):
    o_ref[...] = jnp.maximum(jnp.dot(x_ref[...], w_ref[...]) + b_ref[...], 0.0)


def _embed(x, w, b, blk):
    rows, c = x.shape
    nb = rows // blk
    return pl.pallas_call(
        _embed_body,
        grid=(nb,),
        in_specs=[
            pl.BlockSpec((blk, c), lambda b_: (b_, 0)),
            pl.BlockSpec((c, w.shape[1]), lambda b_: (0, 0)),
            pl.BlockSpec((1, w.shape[1]), lambda b_: (0, 0)),
        ],
        out_specs=pl.BlockSpec((blk, w.shape[1]), lambda b_: (b_, 0)),
        out_shape=jax.ShapeDtypeStruct((rows, w.shape[1]), f32),
    )(x, w, b)


def _msg_body(g_ref, ehx_ref, s_ref, w_ref, o_ref):
    g = g_ref[...]
    ge = jnp.concatenate([g] * K11, axis=1)
    z = ge * jnp.dot(ehx_ref[...], s_ref[...])
    o_ref[...] = jnp.dot(z, w_ref[...])


def _msg(g, ehx, smat, wf):
    blk = 2048
    nb = EP // blk
    return pl.pallas_call(
        _msg_body,
        grid=(nb,),
        in_specs=[
            pl.BlockSpec((blk, DP), lambda b_: (b_, 0)),
            pl.BlockSpec((blk, DEP), lambda b_: (b_, 0)),
            pl.BlockSpec((DEP, KD), lambda b_: (0, 0)),
            pl.BlockSpec((KD, DP), lambda b_: (0, 0)),
        ],
        out_specs=pl.BlockSpec((blk, DP), lambda b_: (b_, 0)),
        out_shape=jax.ShapeDtypeStruct((EP, DP), f32),
    )(g, ehx, smat, wf)


def _upd_body(a0_ref, a1_ref, h_ref, wm_ref, wh_ref, b_ref, o_ref):
    m = jnp.maximum(a0_ref[0] + a1_ref[0], 0.0)
    o_ref[...] = (
        jnp.dot(m, wm_ref[...]) + jnp.dot(h_ref[...], wh_ref[...]) + b_ref[...]
    )


def _upd(agg2, h, wm, wh, b):
    nb = N // RB
    return pl.pallas_call(
        _upd_body,
        grid=(nb,),
        in_specs=[
            pl.BlockSpec((1, RB, DP), lambda b_: (0, b_, 0)),
            pl.BlockSpec((1, RB, DP), lambda b_: (1, b_, 0)),
            pl.BlockSpec((RB, DP), lambda b_: (b_, 0)),
            pl.BlockSpec((DP, DP), lambda b_: (0, 0)),
            pl.BlockSpec((DP, DP), lambda b_: (0, 0)),
            pl.BlockSpec((1, DP), lambda b_: (0, 0)),
        ],
        out_specs=pl.BlockSpec((RB, DP), lambda b_: (b_, 0)),
        out_shape=jax.ShapeDtypeStruct((N, DP), f32),
    )(agg2, agg2, h, wm, wh, b)


def _final_body(hu_ref, xu_ref, hv_ref, xv_ref, bu_ref, bv_ref,
                im_ref, osu_ref, osv_ref, svn_ref, acc_ref):
    i = pl.program_id(0)

    @pl.when(i == 0)
    def _():
        sv = hv_ref[...] + xv_ref[...]
        nv = jnp.sqrt(jnp.sum(sv * sv, axis=1, keepdims=True))
        svn_ref[...] = sv / jnp.maximum(nv, 1e-12)

    su = hu_ref[...] + xu_ref[...]
    nu = jnp.sqrt(jnp.sum(su * su, axis=1, keepdims=True))
    su = su / jnp.maximum(nu, 1e-12)
    svn = svn_ref[...]
    raw = lax.dot_general(su, svn, (((1,), (1,)), ((), ())))
    im = jnp.where(bu_ref[...] == bv_ref[...], raw, 0.0)
    im_ref[...] = im
    osu_ref[:, 0, :] = su
    osu_ref[:, 1, :] = jnp.dot(im, svn)
    contrib = lax.dot_general(im, su, (((0,), (0,)), ((), ())))

    @pl.when(i == 0)
    def _():
        acc_ref[...] = contrib

    @pl.when(i > 0)
    def _():
        acc_ref[...] = acc_ref[...] + contrib

    @pl.when(i == pl.num_programs(0) - 1)
    def _():
        osv_ref[:, 0, :] = svn_ref[...]
        osv_ref[:, 1, :] = acc_ref[...]


def _final(hu, xu, hv, xv, bu, bv):
    nb = N // RBF
    return pl.pallas_call(
        _final_body,
        grid=(nb,),
        in_specs=[
            pl.BlockSpec((RBF, DP), lambda b_: (b_, 0)),
            pl.BlockSpec((RBF, DP), lambda b_: (b_, 0)),
            pl.BlockSpec((N, DP), lambda b_: (0, 0)),
            pl.BlockSpec((N, DP), lambda b_: (0, 0)),
            pl.BlockSpec((RBF, 1), lambda b_: (b_, 0)),
            pl.BlockSpec((1, N), lambda b_: (0, 0)),
        ],
        out_specs=[
            pl.BlockSpec((RBF, N), lambda b_: (b_, 0)),
            pl.BlockSpec((RBF, 2, DP), lambda b_: (b_, 0, 0)),
            pl.BlockSpec((N, 2, DP), lambda b_: (0, 0, 0)),
        ],
        out_shape=[
            jax.ShapeDtypeStruct((N, N), f32),
            jax.ShapeDtypeStruct((N, 2, DP), f32),
            jax.ShapeDtypeStruct((N, 2, DP), f32),
        ],
        scratch_shapes=[
            pltpu.VMEM((N, DP), f32),
            pltpu.VMEM((N, DP), f32),
        ],
    )(hu, xu, hv, xv, bu, bv)


def _sc_mesh():
    return plsc.VectorSubcoreMesh(
        core_axis_name="c", subcore_axis_name="s", num_cores=2, num_subcores=16
    )


def _sc_gather(h_table, src2):
    @functools.partial(
        pl.kernel,
        out_type=jax.ShapeDtypeStruct((EP, DP), f32),
        mesh=_sc_mesh(),
        compiler_params=pltpu.CompilerParams(use_tc_tiling_on_sc=False),
        scratch_types=[
            pltpu.VMEM((NCHW, CH), i32),
            pltpu.VMEM((CH, DP), f32),
            pltpu.VMEM((CH, DP), f32),
            pltpu.VMEM_SHARED((N, DP), f32),
            pltpu.SemaphoreType.DMA,
            pltpu.SemaphoreType.DMA,
        ],
    )
    def gk(h_hbm, src_hbm, out_hbm, idx_v, rows_a, rows_b, tab_sh,
           sem_a, sem_b):
        c = lax.axis_index("c")
        s = lax.axis_index("s")
        wid = s * 2 + c

        @pl.when(s < 15)
        def _():
            pltpu.sync_copy(h_hbm.at[pl.ds(s * 256, 256)],
                            tab_sh.at[pl.ds(s * 256, 256)])

        @pl.when(s == 15)
        def _():
            pltpu.sync_copy(h_hbm.at[pl.ds(3840, 160)],
                            tab_sh.at[pl.ds(3840, 160)])

        pltpu.sync_copy(src_hbm.at[pl.ds(wid * NCHW, NCHW)], idx_v)
        plsc.subcore_barrier()
        bufs = (rows_a, rows_b)
        sems = (sem_a, sem_b)
        cps = [None, None]
        cps[0] = pltpu.async_copy(tab_sh.at[idx_v.at[0]], bufs[0], sems[0])
        for j in range(NCHW):
            if j + 1 < NCHW:
                cps[(j + 1) % 2] = pltpu.async_copy(
                    tab_sh.at[idx_v.at[j + 1]], bufs[(j + 1) % 2],
                    sems[(j + 1) % 2])
            cps[j % 2].wait()
            pltpu.sync_copy(bufs[j % 2],
                            out_hbm.at[pl.ds((wid * NCHW + j) * CH, CH)])

    return gk(h_table, src2)


def _sc_scatter(msg, dst2, zeros_na):
    @functools.partial(
        pl.kernel,
        out_type=jax.ShapeDtypeStruct((2, NA, DP), f32),
        mesh=_sc_mesh(),
        compiler_params=pltpu.CompilerParams(use_tc_tiling_on_sc=False),
        scratch_types=[
            pltpu.VMEM((NCHW, CH), i32),
            pltpu.VMEM((CH, DP), f32),
            pltpu.VMEM((CH, DP), f32),
            pltpu.VMEM_SHARED((NA, DP), f32),
            pltpu.SemaphoreType.DMA,
            pltpu.SemaphoreType.DMA,
        ],
    )
    def sk(msg_hbm, dst_hbm, z_hbm, out_hbm, idx_v, row_a, row_b, agg_sh,
           sem_a, sem_b):
        c = lax.axis_index("c")
        s = lax.axis_index("s")
        wid = s * 2 + c
        pltpu.sync_copy(z_hbm.at[pl.ds(s * RPS, RPS)], agg_sh.at[pl.ds(s * RPS, RPS)])
        pltpu.sync_copy(dst_hbm.at[pl.ds(wid * NCHW, NCHW)], idx_v)
        plsc.subcore_barrier()
        bufs = (row_a, row_b)
        sems = (sem_a, sem_b)
        cps = [None, None]
        cps[0] = pltpu.async_copy(
            msg_hbm.at[pl.ds(wid * NCHW * CH, CH)], bufs[0], sems[0])
        for j in range(NCHW):
            if j + 1 < NCHW:
                cps[(j + 1) % 2] = pltpu.async_copy(
                    msg_hbm.at[pl.ds((wid * NCHW + j + 1) * CH, CH)],
                    bufs[(j + 1) % 2], sems[(j + 1) % 2])
            cps[j % 2].wait()
            pltpu.sync_copy(bufs[j % 2], agg_sh.at[idx_v.at[j]], add=True)
        plsc.subcore_barrier()
        pltpu.sync_copy(
            agg_sh.at[pl.ds(s * RPS, RPS)],
            out_hbm.at[c].at[pl.ds(s * RPS, RPS)],
        )

    return sk(msg, dst2, zeros_na)


def kernel(solute_x, solute_edge_index, solute_edge_attr, solvent_x,
           solvent_edge_index, solvent_edge_attr, solute_batch, solvent_batch,
           su_lin0_W, su_lin0_b, su_e1_W, su_e1_b, su_e2_W, su_e2_b,
           su_msg_W, su_msg_b, sv_lin0_W, sv_lin0_b, sv_e1_W, sv_e1_b,
           sv_e2_W, sv_e2_b, sv_msg_W, sv_msg_b):
    def idxpair(ei):
        ei = ei.astype(i32)
        src = jnp.zeros((EP,), i32).at[:E1].set(ei[0]).reshape(NW * NCHW, CH)
        dst = jnp.full((EP,), TRASH, i32).at[:E1].set(ei[1]).reshape(NW * NCHW, CH)
        return src, dst

    src_u, dst_u = idxpair(solute_edge_index)
    src_v, dst_v = idxpair(solvent_edge_index)

    def padx(x):
        return jnp.zeros((N, DP), f32).at[:, :D].set(x)

    def padea(ea):
        return jnp.zeros((EP, DEP), f32).at[:E1, :DE].set(ea)

    xu, xv = padx(solute_x), padx(solvent_x)
    ea_u, ea_v = padea(solute_edge_attr), padea(solvent_edge_attr)

    def wset(lin0_W, lin0_b, e1_W, e1_b, e2_W, e2_b, msg_W, msg_b):
        w3x = jnp.concatenate(
            [e2_W.reshape(DE, D, D), e2_b.reshape(1, D, D)], axis=0)
        wf = (jnp.zeros((K11, DP, DP), f32).at[:, :D, :D].set(w3x)
              .transpose(0, 2, 1).reshape(KD, DP))
        return dict(
            w0=_pad2(lin0_W, DP, DP), b0=_pad2(lin0_b[None], 1, DP),
            e1=_pad2(e1_W, DEP, DEP),
            b1=_pad2(e1_b[None], 1, DEP).at[0, DE].set(1.0),
            wf=wf,
            wm=_pad2(msg_W[:D], DP, DP), wh=_pad2(msg_W[D:], DP, DP),
            bm=_pad2(msg_b[None], 1, DP),
        )

    wu = wset(su_lin0_W, su_lin0_b, su_e1_W, su_e1_b, su_e2_W, su_e2_b,
              su_msg_W, su_msg_b)
    wv = wset(sv_lin0_W, sv_lin0_b, sv_e1_W, sv_e1_b, sv_e2_W, sv_e2_b,
              sv_msg_W, sv_msg_b)

    smat = jnp.zeros((DEP, KD), f32)
    for _k in range(K11):
        smat = smat.at[_k, _k * DP:(_k + 1) * DP].set(1.0)
    zeros_na = jnp.zeros((NA, DP), f32)
    bu = solute_batch.astype(i32).reshape(N, 1)
    bv = solvent_batch.astype(i32).reshape(1, N)

    hu = _embed(xu, wu["w0"], wu["b0"], RB)
    hv = _embed(xv, wv["w0"], wv["b0"], RB)
    ehx_u = _embed(ea_u, wu["e1"], wu["b1"], 4096)
    ehx_v = _embed(ea_v, wv["e1"], wv["b1"], 4096)
    for _ in range(NSTEP):
        g_u = _sc_gather(hu, src_u)
        g_v = _sc_gather(hv, src_v)
        m_u = _msg(g_u, ehx_u, smat, wu["wf"])
        m_v = _msg(g_v, ehx_v, smat, wv["wf"])
        a_u = _sc_scatter(m_u, dst_u, zeros_na)
        a_v = _sc_scatter(m_v, dst_v, zeros_na)
        hu = _upd(a_u, hu, wu["wm"], wu["wh"], wu["bm"])
        hv = _upd(a_v, hv, wv["wm"], wv["wh"], wv["bm"])

    im, osu, osv = _final(hu, xu, hv, xv, bu, bv)
    out_su = jnp.concatenate([osu[:, 0, :D], osu[:, 1, :D]], axis=1)
    out_sv = jnp.concatenate([osv[:, 0, :D], osv[:, 1, :D]], axis=1)
    return out_su, out_sv, im
